# Initial kernel scaffold; baseline (speedup 1.0000x reference)
import numpy as np
import jax
import jax.numpy as jnp
from jax import lax
from jax.experimental import pallas as pl
from jax.experimental.pallas import tpu as pltpu


_ROTS = ((13, 15, 26, 6), (17, 29, 16, 24))


def _np_threefry2x32(k0, k1, x0, x1):
    m = 0xFFFFFFFF
    ks0, ks1 = k0 & m, k1 & m
    ks2 = (ks0 ^ ks1 ^ 0x1BD11BDA) & m
    ks = (ks0, ks1, ks2)
    x0 = (x0 + ks0) & m
    x1 = (x1 + ks1) & m
    for i in range(5):
        for r in _ROTS[i % 2]:
            x0 = (x0 + x1) & m
            x1 = ((x1 << r) | (x1 >> (32 - r))) & m
            x1 ^= x0
        x0 = (x0 + ks[(i + 1) % 3]) & m
        x1 = (x1 + ks[(i + 2) % 3] + i + 1) & m
    return x0, x1


_K0, _K1 = _np_threefry2x32(0, 42, 0, 1)
_K2 = (_K0 ^ _K1 ^ 0x1BD11BDA) & 0xFFFFFFFF

_F = 4
_BB = 32
_ROW = 32 * 32 * _F


def _u32(v):
    return jnp.uint32(v & 0xFFFFFFFF)


def _threefry_bits(cnt):
    ks = (_u32(_K0), _u32(_K1), _u32(_K2))
    x0 = jnp.full_like(cnt, _u32(_K0))
    x1 = cnt + _u32(_K1)
    for i in range(5):
        for r in _ROTS[i % 2]:
            x0 = x0 + x1
            x1 = lax.shift_left(x1, _u32(r)) | lax.shift_right_logical(
                x1, _u32(32 - r))
            x1 = x1 ^ x0
        x0 = x0 + ks[(i + 1) % 3]
        x1 = x1 + ks[(i + 2) % 3] + _u32(i + 1)
    return x0 ^ x1


def _sample_kernel(x_ref, o_ref):
    x = x_ref[...]
    row = lax.broadcasted_iota(jnp.int32, (_BB, _ROW), 0)
    col = lax.broadcasted_iota(jnp.int32, (_BB, _ROW), 1)
    base = (pl.program_id(0) * _BB) * _ROW
    cnt = lax.bitcast_convert_type(base + row * _ROW + col, jnp.uint32)

    bits = _threefry_bits(cnt)
    fbits = lax.shift_right_logical(bits, _u32(9)) | _u32(0x3F800000)
    f = lax.bitcast_convert_type(fbits, jnp.float32) - jnp.float32(1.0)
    tiny = jnp.float32(np.finfo(np.float32).tiny)
    u = jnp.maximum(tiny, f * (jnp.float32(1.0) - tiny) + tiny)
    y = x + (-jnp.log(-jnp.log(u)))

    m = col & 3
    odd = (m & 1) == 1
    pv = jnp.where(odd, pltpu.roll(y, 1, 1), pltpu.roll(y, -1, 1))
    take_p = (pv > y) | ((pv == y) & odd)
    v1 = jnp.where(take_p, pv, y)
    i1 = jnp.where(take_p, m ^ 1, m)

    hi = m >= 2
    pv2 = jnp.where(hi, pltpu.roll(v1, 2, 1), pltpu.roll(v1, -2, 1))
    pi2 = jnp.where(hi, pltpu.roll(i1, 2, 1), pltpu.roll(i1, -2, 1))
    take_p2 = (pv2 > v1) | ((pv2 == v1) & (pi2 < i1))
    idx = jnp.where(take_p2, pi2, i1)

    o_ref[...] = jnp.where(m == idx, jnp.float32(1.0), jnp.float32(0.0))


def kernel(reconstructed_state_logits):
    logits = reconstructed_state_logits
    squeeze = False
    if logits.ndim == 3:
        logits = logits[None]
        squeeze = True
    B, H, W, Fdim = logits.shape
    x = logits.reshape(B, H * W * Fdim)
    out = pl.pallas_call(
        _sample_kernel,
        grid=(B // _BB,),
        in_specs=[pl.BlockSpec((_BB, _ROW), lambda i: (i, 0))],
        out_specs=pl.BlockSpec((_BB, _ROW), lambda i: (i, 0)),
        out_shape=jax.ShapeDtypeStruct((B, H * W * Fdim), jnp.float32),
    )(x)
    out = out.reshape(B, H, W, Fdim)
    if squeeze:
        out = out[0]
    return out


# speedup vs baseline: 1.2664x; 1.2664x over previous
import numpy as np
import jax
import jax.numpy as jnp
from jax import lax
from jax.experimental import pallas as pl
from jax.experimental.pallas import tpu as pltpu


_ROTS = ((13, 15, 26, 6), (17, 29, 16, 24))


def _np_threefry2x32(k0, k1, x0, x1):
    m = 0xFFFFFFFF
    ks0, ks1 = k0 & m, k1 & m
    ks2 = (ks0 ^ ks1 ^ 0x1BD11BDA) & m
    ks = (ks0, ks1, ks2)
    x0 = (x0 + ks0) & m
    x1 = (x1 + ks1) & m
    for i in range(5):
        for r in _ROTS[i % 2]:
            x0 = (x0 + x1) & m
            x1 = ((x1 << r) | (x1 >> (32 - r))) & m
            x1 ^= x0
        x0 = (x0 + ks[(i + 1) % 3]) & m
        x1 = (x1 + ks[(i + 2) % 3] + i + 1) & m
    return x0, x1


_K0, _K1 = _np_threefry2x32(0, 42, 0, 1)
_K2 = (_K0 ^ _K1 ^ 0x1BD11BDA) & 0xFFFFFFFF

_F = 4
_BB = 32
_ROW = 32 * 32 * _F


def _u32(v):
    return jnp.uint32(v & 0xFFFFFFFF)


def _threefry_bits(cnt):
    ks = (_u32(_K0), _u32(_K1), _u32(_K2))
    x0 = jnp.full_like(cnt, _u32(_K0))
    x1 = cnt + _u32(_K1)
    for i in range(5):
        for r in _ROTS[i % 2]:
            x0 = x0 + x1
            x1 = lax.shift_left(x1, _u32(r)) | lax.shift_right_logical(
                x1, _u32(32 - r))
            x1 = x1 ^ x0
        x0 = x0 + ks[(i + 1) % 3]
        x1 = x1 + ks[(i + 2) % 3] + _u32(i + 1)
    return x0 ^ x1


def _sample_kernel(x_ref, o_ref):
    x = x_ref[...]
    row = lax.broadcasted_iota(jnp.int32, (_BB, _ROW), 0)
    col = lax.broadcasted_iota(jnp.int32, (_BB, _ROW), 1)
    base = (pl.program_id(0) * _BB) * _ROW
    cnt = lax.bitcast_convert_type(base + row * _ROW + col, jnp.uint32)

    bits = _threefry_bits(cnt)
    fbits = lax.shift_right_logical(bits, _u32(9)) | _u32(0x3F800000)
    f = lax.bitcast_convert_type(fbits, jnp.float32) - jnp.float32(1.0)
    tiny = jnp.float32(np.finfo(np.float32).tiny)
    u = jnp.maximum(tiny, f * (jnp.float32(1.0) - tiny) + tiny)
    y = x + (-jnp.log(-jnp.log(u)))

    m = col & 3
    odd = (m & 1) == 1
    pv = jnp.where(odd, pltpu.roll(y, 1, 1), pltpu.roll(y, _ROW - 1, 1))
    take_p = (pv > y) | ((pv == y) & odd)
    v1 = jnp.where(take_p, pv, y)
    i1 = jnp.where(take_p, m ^ 1, m)

    hi = m >= 2
    pv2 = jnp.where(hi, pltpu.roll(v1, 2, 1), pltpu.roll(v1, _ROW - 2, 1))
    pi2 = jnp.where(hi, pltpu.roll(i1, 2, 1), pltpu.roll(i1, _ROW - 2, 1))
    take_p2 = (pv2 > v1) | ((pv2 == v1) & (pi2 < i1))
    idx = jnp.where(take_p2, pi2, i1)

    o_ref[...] = jnp.where(m == idx, jnp.float32(1.0), jnp.float32(0.0))


def kernel(reconstructed_state_logits):
    logits = reconstructed_state_logits
    squeeze = False
    if logits.ndim == 3:
        logits = logits[None]
        squeeze = True
    B, H, W, Fdim = logits.shape
    x = logits.reshape(B, H * W * Fdim)
    out = pl.pallas_call(
        _sample_kernel,
        grid=(B // _BB,),
        in_specs=[pl.BlockSpec((_BB, _ROW), lambda i: (i, 0))],
        out_specs=pl.BlockSpec((_BB, _ROW), lambda i: (i, 0)),
        out_shape=jax.ShapeDtypeStruct((B, H * W * Fdim), jnp.float32),
    )(x)
    out = out.reshape(B, H, W, Fdim)
    if squeeze:
        out = out[0]
    return out
